# baseline (speedup 1.0000x reference)
import functools
import jax
import jax.numpy as jnp
from jax import lax
from jax.experimental import pallas as pl
from jax.experimental.pallas import tpu as pltpu
from jax.experimental.pallas import tpu_sc as plsc

_NB = 3


def _sc_body(B, K, C, H, W, Z,
             x_hbm, m_hbm, z_hbm, out_hbm,
             mv, bufs, rsems, wsems):
    wid = lax.axis_index("s") * 2 + lax.axis_index("c")
    slab = wid // 4
    q = wid % 4
    b = slab // K
    kk = slab % K
    cq = C // 4
    c0 = q * cq
    hh = H // 16
    nch = cq * 16

    def src(i):
        c, h = divmod(i, 16)
        return x_hbm.at[b, kk, c0 + c, pl.ds(h * hh, hh)]

    def dst(i):
        c, h = divmod(i, 16)
        return out_hbm.at[b, (kk * C + c0 + c), pl.ds(h * hh, hh)]

    pltpu.sync_copy(m_hbm, mv)
    sel = mv[pl.ds(slab, 1)][0]

    @pl.when(sel != 0)
    def _copy():
        for i in range(_NB):
            pltpu.make_async_copy(src(i), bufs[i], rsems[i]).start()
        for i in range(nch):
            j = i % _NB
            pltpu.make_async_copy(src(i), bufs[j], rsems[j]).wait()
            pltpu.make_async_copy(bufs[j], dst(i), wsems[j]).start()
            if i + _NB < nch:
                pltpu.make_async_copy(bufs[j], dst(i), wsems[j]).wait()
                pltpu.make_async_copy(src(i + _NB), bufs[j], rsems[j]).start()
            else:
                pltpu.make_async_copy(bufs[j], dst(i), wsems[j]).wait()

    @pl.when(sel == 0)
    def _zero():
        pltpu.sync_copy(z_hbm, bufs[0])
        for i in range(nch):
            pltpu.make_async_copy(bufs[0], dst(i), wsems[0]).start()
        for i in range(nch):
            pltpu.make_async_copy(bufs[0], dst(i), wsems[0]).wait()


def kernel(x, mask):
    B, K, C, H, W, Z = x.shape
    hh = H // 16
    m16 = jnp.pad(mask.reshape(B * K).astype(jnp.int32), (0, 16 - B * K))
    zrow = jnp.zeros((hh, W, Z), jnp.float32)

    mesh = plsc.VectorSubcoreMesh(core_axis_name="c", subcore_axis_name="s")
    fn = functools.partial(
        pl.kernel,
        mesh=mesh,
        out_type=jax.ShapeDtypeStruct((B, K * C, H, W, Z), x.dtype),
        scratch_types=[
            pltpu.VMEM((16,), jnp.int32),
            [pltpu.VMEM((hh, W, Z), jnp.float32) for _ in range(_NB)],
            [pltpu.SemaphoreType.DMA for _ in range(_NB)],
            [pltpu.SemaphoreType.DMA for _ in range(_NB)],
        ],
    )(functools.partial(_sc_body, B, K, C, H, W, Z))
    return fn(x, m16, zrow)
